# baseline (speedup 1.0000x reference)
import jax
import jax.numpy as jnp
from jax.experimental import pallas as pl

V = 1024
C = 64


def _prep_kernel(cb_ref, cbd_ref, cba_ref):
    cb = cb_ref[...]
    csq = jnp.sum(cb * cb, axis=1, keepdims=True)
    hi = csq.astype(jnp.bfloat16)
    r1 = csq - hi.astype(jnp.float32)
    mid = r1.astype(jnp.bfloat16)
    lo = (r1 - mid.astype(jnp.float32)).astype(jnp.bfloat16)
    cbd_ref[...] = jnp.concatenate(
        [(-2.0 * cb).astype(jnp.bfloat16), hi, mid, lo], axis=1)
    cba_ref[...] = jnp.concatenate(
        [cb, jnp.ones((V, 1), jnp.float32)], axis=1).astype(jnp.bfloat16)


def _vq_kernel(x_ref, cbd_ref, cba_ref, xd_ref, counts_ref):
    i = pl.program_id(0)

    nb = x_ref.shape[0]
    xb = jnp.concatenate([x_ref[b] for b in range(nb)], axis=1)

    s = jnp.dot(cbd_ref[...], xb,
                preferred_element_type=jnp.float32)

    minval = jnp.min(s, axis=0, keepdims=True)
    maskb = jnp.where(s <= minval, 1.0, 0.0).astype(jnp.bfloat16)

    yq = jax.lax.dot_general(cba_ref[...], maskb,
                             (((0,), (0,)), ((), ())),
                             preferred_element_type=jnp.float32)
    colsum = yq[C:C + 1]
    scale = jnp.where(colsum == 1.0, 1.0, 1.0 / colsum)
    xd = yq[:C] * scale
    T = xd.shape[1] // nb
    for b in range(nb):
        xd_ref[b] = xd[:, b * T:(b + 1) * T]

    part = jnp.sum(maskb.astype(jnp.float32), axis=1, keepdims=True)
    prev = jnp.where(i == 0, 0.0, counts_ref[...])
    counts_ref[...] = prev + part


def _perp_kernel(counts_ref, perp_ref):
    counts = counts_ref[...]
    prob = counts / jnp.sum(counts)
    ent = jnp.sum(prob * jnp.log(prob + 1e-07),
                  axis=0, keepdims=True)
    perp_ref[...] = jnp.exp(-ent)


def kernel(x, codebook):
    N, width, T = x.shape
    cbd, cba = pl.pallas_call(
        _prep_kernel,
        out_shape=[
            jax.ShapeDtypeStruct((V, C + 3), jnp.bfloat16),
            jax.ShapeDtypeStruct((V, C + 1), jnp.bfloat16),
        ],
    )(codebook)
    xpad = jnp.concatenate(
        [x.astype(jnp.bfloat16),
         jnp.ones((N, 3, T), jnp.bfloat16)], axis=1)
    NB = 4
    xd, counts = pl.pallas_call(
        _vq_kernel,
        grid=(N // NB,),
        in_specs=[
            pl.BlockSpec((NB, width + 3, T), lambda i: (i, 0, 0)),
            pl.BlockSpec((V, C + 3), lambda i: (0, 0)),
            pl.BlockSpec((V, C + 1), lambda i: (0, 0)),
        ],
        out_specs=[
            pl.BlockSpec((NB, width, T), lambda i: (i, 0, 0)),
            pl.BlockSpec((V, 1), lambda i: (0, 0)),
        ],
        out_shape=[
            jax.ShapeDtypeStruct((N, width, T), jnp.float32),
            jax.ShapeDtypeStruct((V, 1), jnp.float32),
        ],
    )(xpad, cbd, cba)
    perp = pl.pallas_call(
        _perp_kernel,
        out_shape=jax.ShapeDtypeStruct((1, 1), jnp.float32),
    )(counts)
    return (xd, perp[0, 0])


# speedup vs baseline: 1.1696x; 1.1696x over previous
import jax
import jax.numpy as jnp
from jax.experimental import pallas as pl
from jax.experimental.pallas import tpu as pltpu

V = 1024
C = 64


def _prep_kernel(cb_ref, cbd_ref, cba_ref):
    cb = cb_ref[...]
    csq = jnp.sum(cb * cb, axis=1, keepdims=True)
    hi = csq.astype(jnp.bfloat16)
    r1 = csq - hi.astype(jnp.float32)
    mid = r1.astype(jnp.bfloat16)
    lo = (r1 - mid.astype(jnp.float32)).astype(jnp.bfloat16)
    cbd_ref[...] = jnp.concatenate(
        [(-2.0 * cb).astype(jnp.bfloat16), hi, mid, lo], axis=1)
    cba_ref[...] = jnp.concatenate(
        [cb, jnp.ones((V, 1), jnp.float32)], axis=1).astype(jnp.bfloat16)


def _vq_kernel(x_ref, cbd_ref, cba_ref, xd_ref, perp_ref, counts_ref):
    i = pl.program_id(0)
    n_steps = pl.num_programs(0)

    nb = x_ref.shape[0]
    xb = jnp.concatenate([x_ref[b] for b in range(nb)],
                         axis=1).astype(jnp.bfloat16)
    W = xb.shape[1]
    xa = jnp.concatenate([xb, jnp.ones((3, W), jnp.bfloat16)], axis=0)

    s = jnp.dot(cbd_ref[...], xa,
                preferred_element_type=jnp.float32)

    minval = jnp.min(s, axis=0, keepdims=True)
    maskf = jnp.where(s <= minval, 1.0, 0.0)
    maskb = maskf.astype(jnp.bfloat16)

    yq = jax.lax.dot_general(cba_ref[...], maskb,
                             (((0,), (0,)), ((), ())),
                             preferred_element_type=jnp.float32)
    colsum = yq[C:C + 1]
    scale = jnp.where(colsum == 1.0, 1.0, 1.0 / colsum)
    xd = yq[:C] * scale
    T = W // nb
    for b in range(nb):
        xd_ref[b] = xd[:, b * T:(b + 1) * T]

    part = jnp.sum(maskf, axis=1, keepdims=True)
    prev = jnp.where(i == 0, 0.0, counts_ref[...])
    counts_ref[...] = prev + part

    @pl.when(i == n_steps - 1)
    def _():
        counts = counts_ref[...]
        prob = counts / jnp.sum(counts)
        ent = jnp.sum(prob * jnp.log(prob + 1e-07),
                      axis=0, keepdims=True)
        perp_ref[...] = jnp.exp(-ent)


def kernel(x, codebook):
    N, width, T = x.shape
    cbd, cba = pl.pallas_call(
        _prep_kernel,
        out_shape=[
            jax.ShapeDtypeStruct((V, C + 3), jnp.bfloat16),
            jax.ShapeDtypeStruct((V, C + 1), jnp.bfloat16),
        ],
    )(codebook)
    NB = 4
    xd, perp = pl.pallas_call(
        _vq_kernel,
        grid=(N // NB,),
        in_specs=[
            pl.BlockSpec((NB, width, T), lambda i: (i, 0, 0)),
            pl.BlockSpec((V, C + 3), lambda i: (0, 0)),
            pl.BlockSpec((V, C + 1), lambda i: (0, 0)),
        ],
        out_specs=[
            pl.BlockSpec((NB, width, T), lambda i: (i, 0, 0)),
            pl.BlockSpec((1, 1), lambda i: (0, 0)),
        ],
        out_shape=[
            jax.ShapeDtypeStruct((N, width, T), jnp.float32),
            jax.ShapeDtypeStruct((1, 1), jnp.float32),
        ],
        scratch_shapes=[pltpu.VMEM((V, 1), jnp.float32)],
    )(x, cbd, cba)
    return (xd, perp[0, 0])


# speedup vs baseline: 1.1702x; 1.0005x over previous
import jax
import jax.numpy as jnp
from jax.experimental import pallas as pl
from jax.experimental.pallas import tpu as pltpu

V = 1024
C = 64


def _prep_kernel(cb_ref, cbd_ref, cba_ref):
    cb = cb_ref[...]
    csq = jnp.sum(cb * cb, axis=1, keepdims=True)
    hi = csq.astype(jnp.bfloat16)
    r1 = csq - hi.astype(jnp.float32)
    mid = r1.astype(jnp.bfloat16)
    lo = (r1 - mid.astype(jnp.float32)).astype(jnp.bfloat16)
    cbd_ref[...] = jnp.concatenate(
        [(-2.0 * cb).astype(jnp.bfloat16), hi, mid, lo], axis=1)
    cba_ref[...] = jnp.concatenate(
        [cb, jnp.ones((V, 1), jnp.float32)], axis=1).astype(jnp.bfloat16)


def _vq_kernel(x_ref, cbd_ref, cba_ref, xd_ref, perp_ref, counts_ref):
    i = pl.program_id(0)
    n_steps = pl.num_programs(0)

    nb = x_ref.shape[0]
    xb = jnp.concatenate([x_ref[b] for b in range(nb)],
                         axis=1).astype(jnp.bfloat16)
    W = xb.shape[1]
    xa = jnp.concatenate([xb, jnp.ones((3, W), jnp.bfloat16)], axis=0)

    s = jnp.dot(cbd_ref[...], xa,
                preferred_element_type=jnp.float32)

    minval = jnp.min(s, axis=0, keepdims=True)
    maskf = jnp.where(s <= minval, 1.0, 0.0)
    maskb = maskf.astype(jnp.bfloat16)

    yq = jax.lax.dot_general(cba_ref[...], maskb,
                             (((0,), (0,)), ((), ())),
                             preferred_element_type=jnp.float32)
    colsum = yq[C:C + 1]
    scale = jnp.where(colsum == 1.0, 1.0, 1.0 / colsum)
    xd = yq[:C] * scale
    T = W // nb
    for b in range(nb):
        xd_ref[b] = xd[:, b * T:(b + 1) * T]

    part = jnp.sum(maskb.astype(jnp.float32), axis=1, keepdims=True)
    prev = jnp.where(i == 0, 0.0, counts_ref[...])
    counts_ref[...] = prev + part

    @pl.when(i == n_steps - 1)
    def _():
        counts = counts_ref[...]
        prob = counts / jnp.sum(counts)
        ent = jnp.sum(prob * jnp.log(prob + 1e-07),
                      axis=0, keepdims=True)
        perp_ref[...] = jnp.exp(-ent)


def kernel(x, codebook):
    N, width, T = x.shape
    cbd, cba = pl.pallas_call(
        _prep_kernel,
        out_shape=[
            jax.ShapeDtypeStruct((V, C + 3), jnp.bfloat16),
            jax.ShapeDtypeStruct((V, C + 1), jnp.bfloat16),
        ],
    )(codebook)
    NB = 4
    xd, perp = pl.pallas_call(
        _vq_kernel,
        grid=(N // NB,),
        in_specs=[
            pl.BlockSpec((NB, width, T), lambda i: (i, 0, 0)),
            pl.BlockSpec((V, C + 3), lambda i: (0, 0)),
            pl.BlockSpec((V, C + 1), lambda i: (0, 0)),
        ],
        out_specs=[
            pl.BlockSpec((NB, width, T), lambda i: (i, 0, 0)),
            pl.BlockSpec((1, 1), lambda i: (0, 0)),
        ],
        out_shape=[
            jax.ShapeDtypeStruct((N, width, T), jnp.float32),
            jax.ShapeDtypeStruct((1, 1), jnp.float32),
        ],
        scratch_shapes=[pltpu.VMEM((V, 1), jnp.float32)],
    )(x, cbd, cba)
    return (xd, perp[0, 0])


# speedup vs baseline: 1.2284x; 1.0498x over previous
import jax
import jax.numpy as jnp
from jax.experimental import pallas as pl
from jax.experimental.pallas import tpu as pltpu

V = 1024
C = 64


def _prep_kernel(cb_ref, cbd_ref, cba_ref):
    cb = cb_ref[...]
    csq = jnp.sum(cb * cb, axis=1, keepdims=True)
    hi = csq.astype(jnp.bfloat16)
    r1 = csq - hi.astype(jnp.float32)
    mid = r1.astype(jnp.bfloat16)
    lo = (r1 - mid.astype(jnp.float32)).astype(jnp.bfloat16)
    cbd_ref[...] = jnp.concatenate(
        [(-2.0 * cb).astype(jnp.bfloat16), hi, mid, lo], axis=1)
    cba_ref[...] = jnp.concatenate(
        [cb, jnp.ones((V, 1), jnp.float32)], axis=1).astype(jnp.bfloat16)


def _vq_kernel(x_ref, cbd_ref, cba_ref, xd_ref, perp_ref, counts_ref):
    i = pl.program_id(0)
    n_steps = pl.num_programs(0)

    nb = x_ref.shape[0]
    xb = jnp.concatenate([x_ref[b] for b in range(nb)],
                         axis=1).astype(jnp.bfloat16)
    W = xb.shape[1]
    xa = jnp.concatenate([xb, jnp.ones((3, W), jnp.bfloat16)], axis=0)

    s = jnp.dot(cbd_ref[...], xa,
                preferred_element_type=jnp.float32)

    minval = jnp.min(s, axis=0, keepdims=True)
    maskf = jnp.where(s <= minval, 1.0, 0.0)
    maskb = maskf.astype(jnp.bfloat16)

    yq = jax.lax.dot_general(cba_ref[...], maskb,
                             (((0,), (0,)), ((), ())),
                             preferred_element_type=jnp.float32)
    colsum = yq[C:C + 1]
    scale = jnp.where(colsum == 1.0, 1.0, 1.0 / colsum)
    xd = yq[:C] * scale
    T = W // nb
    for b in range(nb):
        xd_ref[b] = xd[:, b * T:(b + 1) * T]

    part = jnp.sum(maskb.astype(jnp.float32), axis=1, keepdims=True)
    prev = jnp.where(i == 0, 0.0, counts_ref[...])
    counts_ref[...] = prev + part

    @pl.when(i == n_steps - 1)
    def _():
        counts = counts_ref[...]
        prob = counts / jnp.sum(counts)
        ent = jnp.sum(prob * jnp.log(prob + 1e-07),
                      axis=0, keepdims=True)
        perp_ref[...] = jnp.exp(-ent)


def kernel(x, codebook):
    N, width, T = x.shape
    cbd, cba = pl.pallas_call(
        _prep_kernel,
        out_shape=[
            jax.ShapeDtypeStruct((V, C + 3), jnp.bfloat16),
            jax.ShapeDtypeStruct((V, C + 1), jnp.bfloat16),
        ],
    )(codebook)
    NB = 8
    xd, perp = pl.pallas_call(
        _vq_kernel,
        grid=(N // NB,),
        in_specs=[
            pl.BlockSpec((NB, width, T), lambda i: (i, 0, 0)),
            pl.BlockSpec((V, C + 3), lambda i: (0, 0)),
            pl.BlockSpec((V, C + 1), lambda i: (0, 0)),
        ],
        out_specs=[
            pl.BlockSpec((NB, width, T), lambda i: (i, 0, 0)),
            pl.BlockSpec((1, 1), lambda i: (0, 0)),
        ],
        out_shape=[
            jax.ShapeDtypeStruct((N, width, T), jnp.float32),
            jax.ShapeDtypeStruct((1, 1), jnp.float32),
        ],
        scratch_shapes=[pltpu.VMEM((V, 1), jnp.float32)],
    )(x, cbd, cba)
    return (xd, perp[0, 0])


# speedup vs baseline: 1.2465x; 1.0147x over previous
import jax
import jax.numpy as jnp
from jax.experimental import pallas as pl
from jax.experimental.pallas import tpu as pltpu

V = 1024
C = 64


def _prep_kernel(cb_ref, cbd_ref, cba_ref):
    cb = cb_ref[...]
    csq = jnp.sum(cb * cb, axis=1, keepdims=True)
    hi = csq.astype(jnp.bfloat16)
    r1 = csq - hi.astype(jnp.float32)
    mid = r1.astype(jnp.bfloat16)
    lo = (r1 - mid.astype(jnp.float32)).astype(jnp.bfloat16)
    cbd_ref[...] = jnp.concatenate(
        [(-2.0 * cb).astype(jnp.bfloat16), hi, mid, lo], axis=1)
    cba_ref[...] = jnp.concatenate(
        [cb, jnp.ones((V, 1), jnp.float32)], axis=1).astype(jnp.bfloat16)


def _vq_kernel(x_ref, cbd_ref, cba_ref, xd_ref, perp_ref, counts_ref):
    i = pl.program_id(0)
    n_steps = pl.num_programs(0)

    nb = x_ref.shape[0]
    xb = jnp.concatenate([x_ref[b] for b in range(nb)],
                         axis=1).astype(jnp.bfloat16)
    W = xb.shape[1]
    xa = jnp.concatenate([xb, jnp.ones((3, W), jnp.bfloat16)], axis=0)

    s = jnp.dot(cbd_ref[...], xa,
                preferred_element_type=jnp.float32)

    minval = jnp.min(s, axis=0, keepdims=True)
    maskf = jnp.where(s <= minval, 1.0, 0.0)
    maskb = maskf.astype(jnp.bfloat16)

    yq = jax.lax.dot_general(cba_ref[...], maskb,
                             (((0,), (0,)), ((), ())),
                             preferred_element_type=jnp.float32)
    colsum = yq[C:C + 1]
    scale = jnp.where(colsum == 1.0, 1.0, 1.0 / colsum)
    xd = yq[:C] * scale
    T = W // nb
    for b in range(nb):
        xd_ref[b] = xd[:, b * T:(b + 1) * T]

    part = jnp.sum(maskb.astype(jnp.float32), axis=1, keepdims=True)
    prev = jnp.where(i == 0, 0.0, counts_ref[...])
    counts_ref[...] = prev + part

    @pl.when(i == n_steps - 1)
    def _():
        counts = counts_ref[...]
        prob = counts / jnp.sum(counts)
        ent = jnp.sum(prob * jnp.log(prob + 1e-07),
                      axis=0, keepdims=True)
        perp_ref[...] = jnp.exp(-ent)


def kernel(x, codebook):
    N, width, T = x.shape
    cbd, cba = pl.pallas_call(
        _prep_kernel,
        out_shape=[
            jax.ShapeDtypeStruct((V, C + 3), jnp.bfloat16),
            jax.ShapeDtypeStruct((V, C + 1), jnp.bfloat16),
        ],
    )(codebook)
    NB = 16
    xd, perp = pl.pallas_call(
        _vq_kernel,
        grid=(N // NB,),
        in_specs=[
            pl.BlockSpec((NB, width, T), lambda i: (i, 0, 0)),
            pl.BlockSpec((V, C + 3), lambda i: (0, 0)),
            pl.BlockSpec((V, C + 1), lambda i: (0, 0)),
        ],
        out_specs=[
            pl.BlockSpec((NB, width, T), lambda i: (i, 0, 0)),
            pl.BlockSpec((1, 1), lambda i: (0, 0)),
        ],
        out_shape=[
            jax.ShapeDtypeStruct((N, width, T), jnp.float32),
            jax.ShapeDtypeStruct((1, 1), jnp.float32),
        ],
        scratch_shapes=[pltpu.VMEM((V, 1), jnp.float32)],
    )(x, cbd, cba)
    return (xd, perp[0, 0])


# speedup vs baseline: 1.3240x; 1.0621x over previous
import jax
import jax.numpy as jnp
from jax.experimental import pallas as pl
from jax.experimental.pallas import tpu as pltpu

V = 1024
C = 64


def _vq_kernel(x_ref, cb_ref, xd_ref, perp_ref,
               counts_ref, cbd_ref, cba_ref):
    i = pl.program_id(0)
    n_steps = pl.num_programs(0)

    @pl.when(i == 0)
    def _():
        cb = cb_ref[...]
        csq = jnp.sum(cb * cb, axis=1, keepdims=True)
        hi = csq.astype(jnp.bfloat16)
        r1 = csq - hi.astype(jnp.float32)
        mid = r1.astype(jnp.bfloat16)
        lo = (r1 - mid.astype(jnp.float32)).astype(jnp.bfloat16)
        cbd_ref[...] = jnp.concatenate(
            [(-2.0 * cb).astype(jnp.bfloat16), hi, mid, lo], axis=1)
        cba_ref[...] = jnp.concatenate(
            [cb, jnp.ones((V, 1), jnp.float32)], axis=1).astype(jnp.bfloat16)

    nb = x_ref.shape[0]
    xb = jnp.concatenate([x_ref[b] for b in range(nb)],
                         axis=1).astype(jnp.bfloat16)
    W = xb.shape[1]
    xa = jnp.concatenate([xb, jnp.ones((3, W), jnp.bfloat16)], axis=0)

    s = jnp.dot(cbd_ref[...], xa,
                preferred_element_type=jnp.float32)

    minval = jnp.min(s, axis=0, keepdims=True)
    maskf = jnp.where(s <= minval, 1.0, 0.0)
    maskb = maskf.astype(jnp.bfloat16)

    yq = jax.lax.dot_general(cba_ref[...], maskb,
                             (((0,), (0,)), ((), ())),
                             preferred_element_type=jnp.float32)
    colsum = yq[C:C + 1]
    scale = jnp.where(colsum == 1.0, 1.0, 1.0 / colsum)
    xd = yq[:C] * scale
    T = W // nb
    for b in range(nb):
        xd_ref[b] = xd[:, b * T:(b + 1) * T]

    part = jnp.sum(maskb.astype(jnp.float32), axis=1, keepdims=True)
    prev = jnp.where(i == 0, 0.0, counts_ref[...])
    counts_ref[...] = prev + part

    @pl.when(i == n_steps - 1)
    def _():
        counts = counts_ref[...]
        prob = counts / jnp.sum(counts)
        ent = jnp.sum(prob * jnp.log(prob + 1e-07),
                      axis=0, keepdims=True)
        perp_ref[...] = jnp.exp(-ent)


def kernel(x, codebook):
    N, width, T = x.shape
    NB = 16
    xd, perp = pl.pallas_call(
        _vq_kernel,
        grid=(N // NB,),
        in_specs=[
            pl.BlockSpec((NB, width, T), lambda i: (i, 0, 0)),
            pl.BlockSpec((V, C), lambda i: (0, 0)),
        ],
        out_specs=[
            pl.BlockSpec((NB, width, T), lambda i: (i, 0, 0)),
            pl.BlockSpec((1, 1), lambda i: (0, 0)),
        ],
        out_shape=[
            jax.ShapeDtypeStruct((N, width, T), jnp.float32),
            jax.ShapeDtypeStruct((1, 1), jnp.float32),
        ],
        scratch_shapes=[
            pltpu.VMEM((V, 1), jnp.float32),
            pltpu.VMEM((V, C + 3), jnp.bfloat16),
            pltpu.VMEM((V, C + 1), jnp.bfloat16),
        ],
    )(x, codebook)
    return (xd, perp[0, 0])
